# Initial kernel scaffold; baseline (speedup 1.0000x reference)
import jax
import jax.numpy as jnp
from jax.experimental import pallas as pl


def kernel(node_feat, edge_feat, edge_index, W0, b0, W1, b1, W2, b2, W3, b3, Wc1, bc1, Wc2, bc2):
    raise NotImplementedError("write your pallas kernel here")


# speedup vs baseline: 2.6768x; 2.6768x over previous
import jax
import jax.numpy as jnp
from jax import lax
from jax.experimental import pallas as pl
from jax.experimental.pallas import tpu as pltpu
from jax.experimental.pallas import tpu_sc as plsc

N_NODES = 10000
N_PAD = 10112
ROWS_PER_SUB = N_PAD // 16
N_EDGES = 320000
CHUNK = 128
NCH = N_EDGES // CHUNK
NW = 32
GROUPS_PER_W = 10
NCHP = NW * GROUPS_PER_W * 8


def _mesh():
    return plsc.VectorSubcoreMesh(core_axis_name="c", subcore_axis_name="s")


def _sc_spmm(d, nrows):
    def body(tab_hbm, src_hbm, dst_hbm, z_hbm, out_hbm,
             src8, dst8, rows, zbuf, acc_sh, sem):
        c = lax.axis_index("c")
        s = lax.axis_index("s")
        wid = s * 2 + c
        sl = pl.ds(s * ROWS_PER_SUB, ROWS_PER_SUB)
        pltpu.sync_copy(z_hbm, zbuf)
        pltpu.sync_copy(zbuf, acc_sh.at[sl])
        plsc.subcore_barrier()

        def outer(t, carry):
            st = wid * (GROUPS_PER_W * 8) + 8 * t
            pltpu.sync_copy(src_hbm.at[pl.ds(st, 8)], src8)
            pltpu.sync_copy(dst_hbm.at[pl.ds(st, 8)], dst8)
            for k in range(8):

                @pl.when(st + k < NCH)
                def _():
                    pltpu.async_copy(tab_hbm.at[src8.at[k]], rows, sem).wait()
                    pltpu.sync_copy(rows, acc_sh.at[dst8.at[k]], add=True)

            return carry

        lax.fori_loop(0, GROUPS_PER_W, outer, 0)
        plsc.subcore_barrier()
        pltpu.sync_copy(acc_sh.at[sl], zbuf)
        pltpu.sync_copy(zbuf, out_hbm.at[c, sl])

    return pl.kernel(
        body,
        out_type=jax.ShapeDtypeStruct((2, N_PAD, d), jnp.float32),
        mesh=_mesh(),
        compiler_params=pltpu.CompilerParams(use_tc_tiling_on_sc=False),
        scratch_types=[
            pltpu.VMEM((8, CHUNK), jnp.int32),
            pltpu.VMEM((8, CHUNK), jnp.int32),
            pltpu.VMEM((CHUNK, d), jnp.float32),
            pltpu.VMEM((ROWS_PER_SUB, d), jnp.float32),
            pltpu.VMEM_SHARED((N_PAD, d), jnp.float32),
            pltpu.SemaphoreType.DMA,
        ],
    )


BLK = 2000


def _row_spec(d):
    return pl.BlockSpec((BLK, d), lambda i: (i, 0))


def _pair_spec(d):
    return pl.BlockSpec((2, BLK, d), lambda i: (0, i, 0))


def _bcast_spec(shape):
    return pl.BlockSpec(shape, lambda i: tuple(0 for _ in shape))


def _tc_build(nf_ref, e2np_ref, degp_ref, x_ref, deg_ref):
    x_ref[...] = jnp.concatenate(
        [nf_ref[...], e2np_ref[0] + e2np_ref[1]], axis=1)
    deg_ref[...] = (degp_ref[0, :, 0:1] + degp_ref[1, :, 0:1]) + 1.0


def _call_build(nf, e2np, degp):
    return pl.pallas_call(
        _tc_build,
        grid=(5,),
        in_specs=[_row_spec(128), _pair_spec(16), _pair_spec(8)],
        out_specs=(_row_spec(144), _row_spec(1)),
        out_shape=(jax.ShapeDtypeStruct((N_NODES, 144), jnp.float32),
                   jax.ShapeDtypeStruct((N_NODES, 1), jnp.float32)),
    )(nf, e2np, degp)


def _tc_layer(accp_ref, cur_ref, wt_ref, b_ref, deg_ref, out_ref):
    n2n = accp_ref[0] + accp_ref[1] + cur_ref[...]
    lin = jnp.dot(n2n, wt_ref[...], preferred_element_type=jnp.float32)
    out_ref[...] = jnp.tanh((lin + b_ref[...]) / deg_ref[...])


def _tc_layer1(accpa_ref, accpb_ref, cur_ref, wt_ref, b_ref, deg_ref, out_ref):
    n2n = jnp.concatenate(
        [accpa_ref[0] + accpa_ref[1], accpb_ref[0] + accpb_ref[1]],
        axis=1) + cur_ref[...]
    lin = jnp.dot(n2n, wt_ref[...], preferred_element_type=jnp.float32)
    out_ref[...] = jnp.tanh((lin + b_ref[...]) / deg_ref[...])


def _call_layer1(accpa, accpb, cur, wt, b, deg):
    return pl.pallas_call(
        _tc_layer1,
        grid=(5,),
        in_specs=[_pair_spec(72), _pair_spec(72), _row_spec(144),
                  _bcast_spec((144, 32)), _bcast_spec((1, 32)), _row_spec(1)],
        out_specs=_row_spec(32),
        out_shape=jax.ShapeDtypeStruct((N_NODES, 32), jnp.float32),
    )(accpa, accpb, cur, wt, b, deg)


def _call_layer(accp, cur, wt, b, deg, din, dout):
    return pl.pallas_call(
        _tc_layer,
        grid=(5,),
        in_specs=[_pair_spec(din), _row_spec(din), _bcast_spec((din, dout)),
                  _bcast_spec((1, dout)), _row_spec(1)],
        out_specs=_row_spec(dout),
        out_shape=jax.ShapeDtypeStruct((N_NODES, dout), jnp.float32),
    )(accp, cur, wt, b, deg)


def _tc_topk(z_ref, idx_ref, val_ref):
    z = z_ref[...]
    pos = (lax.broadcasted_iota(jnp.int32, (8, 1250), 0) * 1250
           + lax.broadcasted_iota(jnp.int32, (8, 1250), 1))
    col = lax.broadcasted_iota(jnp.int32, (1, 32), 1)

    def body(t, carry):
        z, idxv, valv = carry
        m = jnp.max(z)
        idx = jnp.min(jnp.where(z == m, pos, jnp.int32(N_NODES)))
        tcol = (col == t)
        idxv = jnp.where(tcol, idx, idxv)
        valv = jnp.where(tcol, m, valv)
        z = jnp.where(pos == idx, jnp.float32(-2.0), z)
        return z, idxv, valv

    _, idxv, valv = lax.fori_loop(
        0, 30, body,
        (z, jnp.full((1, 32), N_NODES, jnp.int32),
         jnp.zeros((1, 32), jnp.float32)))
    idx_ref[...] = idxv
    val_ref[...] = valv


def _tmm(a, b):
    return lax.dot_general(a, b, (((0,), (0,)), ((), ())),
                           preferred_element_type=jnp.float32)


def _tc_head(cur0_ref, cur1_ref, cur2_ref, idx_ref, val_ref,
             v1t_ref, bc1_ref, aev_ref, aod_ref, sj_ref, w2j_ref, bc2_ref,
             out_ref):
    iota = lax.broadcasted_iota(jnp.int32, (N_NODES, 1), 0)
    sel = (iota == idx_ref[...]).astype(jnp.float32)
    vcol = _tmm(val_ref[...], jnp.ones((1, 1), jnp.float32))
    sp = jnp.concatenate(
        [_tmm(sel, cur0_ref[...]), _tmm(sel, cur1_ref[...]),
         _tmm(sel, cur2_ref[...]), vcol], axis=1)
    c1 = jax.nn.relu(
        jnp.dot(sp, v1t_ref[...], preferred_element_type=jnp.float32)
        + bc1_ref[...])
    p1 = jnp.maximum(
        jnp.dot(aev_ref[...], c1, preferred_element_type=jnp.float32),
        jnp.dot(aod_ref[...], c1, preferred_element_type=jnp.float32))
    acc = jnp.zeros((11, 32), jnp.float32)
    for j in range(5):
        acc = acc + jnp.dot(
            jnp.dot(sj_ref[j], p1, preferred_element_type=jnp.float32),
            w2j_ref[j], preferred_element_type=jnp.float32)
    q = jax.nn.relu(acc + bc2_ref[...])
    out_ref[...] = jax.nn.relu(jnp.max(q, axis=0, keepdims=True))


def _tc_call(fn, out_shapes, *args):
    return pl.pallas_call(
        fn,
        out_shape=out_shapes,
    )(*args)


def _pad_chunks(v):
    return jnp.pad(v, (0, NCHP * CHUNK - N_EDGES)).reshape(NCHP, CHUNK)


def kernel(node_feat, edge_feat, edge_index, W0, b0, W1, b1, W2, b2, W3, b3,
           Wc1, bc1, Wc2, bc2):
    f32 = jnp.float32
    src_r = _pad_chunks(edge_index[0])
    dst_r = _pad_chunks(edge_index[1])
    eid_r = _pad_chunks(jnp.arange(N_EDGES, dtype=jnp.int32))
    ones_tab = jnp.ones((8, 8), f32)
    z8 = jnp.zeros((ROWS_PER_SUB, 8), f32)
    z16 = jnp.zeros((ROWS_PER_SUB, 16), f32)
    z32 = jnp.zeros((ROWS_PER_SUB, 32), f32)
    z72 = jnp.zeros((ROWS_PER_SUB, 72), f32)

    e2np = _sc_spmm(16, N_EDGES)(edge_feat, eid_r, dst_r, z16)
    degp = _sc_spmm(8, 8)(ones_tab, jnp.zeros_like(src_r), dst_r, z8)

    x, deg = _call_build(node_feat, e2np, degp)

    spmm32 = _sc_spmm(32, N_NODES)
    spmm72 = _sc_spmm(72, N_NODES)
    xa = x[:, :72]
    xb = x[:, 72:]
    accxa = spmm72(xa, src_r, dst_r, z72)
    accxb = spmm72(xb, src_r, dst_r, z72)
    cur0 = _call_layer1(accxa, accxb, x, W0.T, b0.reshape(1, 32), deg)

    acc0 = spmm32(cur0, src_r, dst_r, z32)
    cur1 = _call_layer(acc0, cur0, W1.T, b1.reshape(1, 32), deg, 32, 32)

    acc1 = spmm32(cur1, src_r, dst_r, z32)
    cur2 = _call_layer(acc1, cur1, W2.T, b2.reshape(1, 32), deg, 32, 32)

    acc2 = spmm32(cur2, src_r, dst_r, z32)
    cur3 = _call_layer(acc2, cur2, W3.T, b3.reshape(1, 1), deg, 32, 1)

    zr = cur3[:, 0].reshape(8, 1250)
    idxv, valv = _tc_call(
        _tc_topk,
        (jax.ShapeDtypeStruct((1, 32), jnp.int32),
         jax.ShapeDtypeStruct((1, 32), f32)),
        zr)

    v1t = Wc1[:, 0, :].T
    r15 = jnp.arange(15)[:, None]
    c32 = jnp.arange(32)[None, :]
    aev = (c32 == 2 * r15).astype(f32)
    aod = (c32 == 2 * r15 + 1).astype(f32)
    r11 = jnp.arange(11)[:, None]
    c15 = jnp.arange(15)[None, :]
    sj = jnp.stack([(c15 == r11 + j).astype(f32) for j in range(5)])
    w2j = jnp.transpose(Wc2, (2, 1, 0))

    out = _tc_call(
        _tc_head,
        jax.ShapeDtypeStruct((1, 32), f32),
        cur0, cur1, cur2, idxv, valv,
        v1t, bc1.reshape(1, 16), aev, aod, sj, w2j, bc2.reshape(1, 32))
    return out
